# baseline (speedup 1.0000x reference)
import functools

import jax
import jax.numpy as jnp
from jax import lax
from jax.experimental import pallas as pl
from jax.experimental.pallas import tpu as pltpu
from jax.experimental.pallas import tpu_sc as plsc

N = 32768
M = 16
D0 = 128
D1 = 256

BLK = 8192
NB = N // BLK
ROWS = BLK // 128

NSC = 16
H = N // 2
NBH = H // BLK
CHA = H // NSC
CHB = N // NSC
L = 16


def _tc_body(f0, f1, w0, w1, b, p_out, e_out, gmax_out, mscr):
    i = pl.program_id(0)
    dn = (((0,), (1,)), ((), ()))
    acc = lax.dot_general(w0[...], f0[...], dn,
                          preferred_element_type=jnp.float32)
    acc = acc + lax.dot_general(w1[...], f1[...], dn,
                                preferred_element_type=jnp.float32)
    e = acc[0:1, :] + b[0:1, 0:1]
    p = acc[1:2, :]
    p_out[...] = p.reshape(1, 1, BLK)
    e_out[...] = e.reshape(1, 1, BLK)
    bm = jnp.max(p)

    @pl.when(i == 0)
    def _init():
        mscr[...] = jnp.full((1, 128), -jnp.inf, jnp.float32)

    mscr[...] = jnp.maximum(mscr[...], bm)

    @pl.when(i == NBH - 1)
    def _fin():
        gmax_out[...] = mscr[...]


def _tc_pass1(feat0, feat1, w0, w1, bvec, off):
    return pl.pallas_call(
        _tc_body,
        grid=(NBH,),
        in_specs=[
            pl.BlockSpec((BLK, D0), lambda i: (i + off, 0)),
            pl.BlockSpec((BLK, D1), lambda i: (i + off, 0)),
            pl.BlockSpec((D0, 2), lambda i: (0, 0)),
            pl.BlockSpec((D1, 2), lambda i: (0, 0)),
            pl.BlockSpec((8, 128), lambda i: (0, 0)),
        ],
        out_specs=[
            pl.BlockSpec((1, 1, BLK), lambda i: (i, 0, 0)),
            pl.BlockSpec((1, 1, BLK), lambda i: (i, 0, 0)),
            pl.BlockSpec((1, 128), lambda i: (0, 0)),
        ],
        out_shape=[
            jax.ShapeDtypeStruct((NBH, 1, BLK), jnp.float32),
            jax.ShapeDtypeStruct((NBH, 1, BLK), jnp.float32),
            jax.ShapeDtypeStruct((1, 128), jnp.float32),
        ],
        scratch_shapes=[pltpu.VMEM((1, 128), jnp.float32)],
    )(feat0, feat1, w0, w1, bvec)


def _phase_a(p_v, e_v, mol_v, rel_v, maxvec, acc_v, n16):
    z16 = jnp.zeros((L,), jnp.float32)
    for bk in range(4):
        acc_v[pl.ds(bk * 2 * L, L)] = z16
        acc_v[pl.ds(bk * 2 * L + L, L)] = z16

    @plsc.parallel_loop(0, n16, unroll=4)
    def body_a(c):
        sl = pl.ds(c * L, L)
        off = (c % 4) * 2 * M
        mol16 = mol_v[sl] + off
        rel = jnp.exp(p_v[sl] - maxvec)
        rel_v[sl] = rel
        plsc.addupdate_scatter(acc_v, [mol16], rel)
        plsc.addupdate_scatter(acc_v, [mol16 + M], rel * e_v[sl])

    zsum = ((acc_v[pl.ds(0, L)] + acc_v[pl.ds(2 * L, L)])
            + (acc_v[pl.ds(4 * L, L)] + acc_v[pl.ds(6 * L, L)]))
    wsum = ((acc_v[pl.ds(L, L)] + acc_v[pl.ds(3 * L, L)])
            + (acc_v[pl.ds(5 * L, L)] + acc_v[pl.ds(7 * L, L)]))
    acc_v[pl.ds(0, L)] = zsum
    acc_v[pl.ds(L, L)] = wsum


def _sc_a_body(p_hbm, e_hbm, mol_hbm, gmax_hbm,
               rel_hbm, parts_hbm,
               p_v, e_v, mol_v, rel_v, gmax_v, acc_v, sem):
    sid = lax.axis_index("s")
    base = sid * CHA

    c1 = pltpu.async_copy(p_hbm.at[pl.ds(base, CHA)], p_v, sem)
    c2 = pltpu.async_copy(e_hbm.at[pl.ds(base, CHA)], e_v, sem)
    c3 = pltpu.async_copy(mol_hbm.at[pl.ds(base, CHA)], mol_v, sem)
    c4 = pltpu.async_copy(gmax_hbm.at[pl.ds(0, L)], gmax_v, sem)
    c1.wait()
    c2.wait()
    c3.wait()
    c4.wait()

    _phase_a(p_v, e_v, mol_v, rel_v, gmax_v[...], acc_v, CHA // L)

    pltpu.sync_copy(rel_v, rel_hbm.at[pl.ds(base, CHA)])
    pltpu.sync_copy(acc_v.at[pl.ds(0, 2 * M)],
                    parts_hbm.at[pl.ds(sid * 2 * M, 2 * M)])


def _sc_a(p0_flat, e0_flat, mol_index, max0_flat):
    mesh = plsc.VectorSubcoreMesh(core_axis_name="c", subcore_axis_name="s",
                                  num_cores=1)
    fn = pl.kernel(
        _sc_a_body,
        out_type=[
            jax.ShapeDtypeStruct((H,), jnp.float32),
            jax.ShapeDtypeStruct((NSC * 2 * M,), jnp.float32),
        ],
        mesh=mesh,
        compiler_params=pltpu.CompilerParams(needs_layout_passes=False),
        scratch_types=[
            pltpu.VMEM((CHA,), jnp.float32),
            pltpu.VMEM((CHA,), jnp.float32),
            pltpu.VMEM((CHA,), jnp.int32),
            pltpu.VMEM((CHA,), jnp.float32),
            pltpu.VMEM((L,), jnp.float32),
            pltpu.VMEM((8 * L,), jnp.float32),
            pltpu.SemaphoreType.DMA,
        ],
    )
    return fn(p0_flat, e0_flat, mol_index, max0_flat)


def _sc_b_body(p1_hbm, e1_hbm, e0_hbm, rel0_hbm, mol_hbm, max0_hbm, max1_hbm,
               parts0_hbm,
               prob_hbm, ae_hbm, contrib_hbm, parts1_hbm, rel1_hbm,
               pa_v, ea_v, mola_v, rela_v, max0_v, max1_v, acc_v,
               eb_v, molb_v, relb_v, prob_v, ae_v, invz_v, red_v, out16_v,
               sem):
    sid = lax.axis_index("s")
    basea = sid * CHA
    baseb = sid * CHB

    c1 = pltpu.async_copy(p1_hbm.at[pl.ds(basea, CHA)], pa_v, sem)
    c2 = pltpu.async_copy(e1_hbm.at[pl.ds(basea, CHA)], ea_v, sem)
    c3 = pltpu.async_copy(mol_hbm.at[pl.ds(H + basea, CHA)], mola_v, sem)
    c4 = pltpu.async_copy(max0_hbm.at[pl.ds(0, L)], max0_v, sem)
    c5 = pltpu.async_copy(max1_hbm.at[pl.ds(0, L)], max1_v, sem)
    c1.wait()
    c2.wait()
    c3.wait()
    c4.wait()
    c5.wait()

    _phase_a(pa_v, ea_v, mola_v, rela_v, max1_v[...], acc_v, CHA // L)
    pltpu.sync_copy(rela_v, rel1_hbm.at[pl.ds(basea, CHA)])
    pltpu.sync_copy(acc_v.at[pl.ds(0, 2 * M)],
                    parts1_hbm.at[pl.ds(sid * 2 * M, 2 * M)])
    plsc.subcore_barrier()

    c6 = pltpu.async_copy(parts0_hbm, red_v.at[pl.ds(0, NSC * 2 * M)], sem)
    c7 = pltpu.async_copy(parts1_hbm,
                          red_v.at[pl.ds(NSC * 2 * M, NSC * 2 * M)], sem)
    c6.wait()
    c7.wait()

    def red_zw(j, zw):
        z0, w0, z1, w1 = zw
        return (z0 + red_v[pl.ds(j * 2 * M, M)],
                w0 + red_v[pl.ds(j * 2 * M + M, M)],
                z1 + red_v[pl.ds(NSC * 2 * M + j * 2 * M, M)],
                w1 + red_v[pl.ds(NSC * 2 * M + j * 2 * M + M, M)])

    zz = jnp.zeros((L,), jnp.float32)
    z0, w0, z1, w1 = lax.fori_loop(0, NSC, red_zw, (zz, zz, zz, zz))
    g = jnp.maximum(max0_v[...], max1_v[...])
    s0 = jnp.exp(max0_v[...] - g)
    s1 = jnp.exp(max1_v[...] - g)
    zt = s0 * z0 + s1 * z1
    wt = s0 * w0 + s1 * w1
    half0 = sid < (NSC // 2)
    sh = jnp.where(half0, s0, s1)
    invz_v[...] = sh / zt

    @pl.when(sid == 0)
    def _final():
        out16_v[...] = jnp.where(zt > 0.0, wt / zt, 0.0)
        pltpu.sync_copy(out16_v, contrib_hbm)

    relbase = jnp.where(half0, baseb, baseb - H)
    c8 = pltpu.async_copy(mol_hbm.at[pl.ds(baseb, CHB)], molb_v, sem)

    @pl.when(half0)
    def _ld0():
        ca = pltpu.async_copy(rel0_hbm.at[pl.ds(relbase, CHB)], relb_v, sem)
        cb = pltpu.async_copy(e0_hbm.at[pl.ds(relbase, CHB)], eb_v, sem)
        ca.wait()
        cb.wait()

    @pl.when(jnp.logical_not(half0))
    def _ld1():
        ca = pltpu.async_copy(rel1_hbm.at[pl.ds(relbase, CHB)], relb_v, sem)
        cb = pltpu.async_copy(e1_hbm.at[pl.ds(relbase, CHB)], eb_v, sem)
        ca.wait()
        cb.wait()

    c8.wait()

    @plsc.parallel_loop(0, CHB // L, unroll=4)
    def body_b(c):
        sl = pl.ds(c * L, L)
        mol16 = molb_v[sl]
        izg = plsc.load_gather(invz_v, [mol16])
        prob = relb_v[sl] * izg
        prob_v[sl] = prob
        ae_v[sl] = prob * eb_v[sl]

    pltpu.sync_copy(prob_v, prob_hbm.at[pl.ds(baseb, CHB)])
    pltpu.sync_copy(ae_v, ae_hbm.at[pl.ds(baseb, CHB)])


def _sc_b(p1_flat, e1_flat, e0_flat, rel0, mol_index, max0_flat, max1_flat,
          parts0):
    mesh = plsc.VectorSubcoreMesh(core_axis_name="c", subcore_axis_name="s",
                                  num_cores=1)
    fn = pl.kernel(
        _sc_b_body,
        out_type=[
            jax.ShapeDtypeStruct((N,), jnp.float32),
            jax.ShapeDtypeStruct((N,), jnp.float32),
            jax.ShapeDtypeStruct((M,), jnp.float32),
            jax.ShapeDtypeStruct((NSC * 2 * M,), jnp.float32),
            jax.ShapeDtypeStruct((H,), jnp.float32),
        ],
        mesh=mesh,
        compiler_params=pltpu.CompilerParams(needs_layout_passes=False),
        scratch_types=[
            pltpu.VMEM((CHA,), jnp.float32),
            pltpu.VMEM((CHA,), jnp.float32),
            pltpu.VMEM((CHA,), jnp.int32),
            pltpu.VMEM((CHA,), jnp.float32),
            pltpu.VMEM((L,), jnp.float32),
            pltpu.VMEM((L,), jnp.float32),
            pltpu.VMEM((8 * L,), jnp.float32),
            pltpu.VMEM((CHB,), jnp.float32),
            pltpu.VMEM((CHB,), jnp.int32),
            pltpu.VMEM((CHB,), jnp.float32),
            pltpu.VMEM((CHB,), jnp.float32),
            pltpu.VMEM((CHB,), jnp.float32),
            pltpu.VMEM((L,), jnp.float32),
            pltpu.VMEM((2 * NSC * 2 * M,), jnp.float32),
            pltpu.VMEM((L,), jnp.float32),
            pltpu.SemaphoreType.DMA,
        ],
    )
    return fn(p1_flat, e1_flat, e0_flat, rel0, mol_index, max0_flat,
              max1_flat, parts0)


def kernel(feat0, feat1, W_e0, W_e1, b_e1, W_p0, W_p1, mol_index, atom_index,
           n_molecules, n_atoms_max):
    w0 = jnp.concatenate([W_e0, W_p0], axis=1)
    w1 = jnp.concatenate([W_e1, W_p1], axis=1)
    bvec = jnp.broadcast_to(b_e1.reshape(1, 1), (8, 128))
    p0, e0, max0 = _tc_pass1(feat0, feat1, w0, w1, bvec, 0)
    p1, e1, max1 = _tc_pass1(feat0, feat1, w0, w1, bvec, NBH)
    p0f, e0f = p0.reshape(H), e0.reshape(H)
    p1f, e1f = p1.reshape(H), e1.reshape(H)
    rel0, parts0 = _sc_a(p0f, e0f, mol_index, max0.reshape(128))
    prob_f, ae_f, contrib, _pp, _rr = _sc_b(
        p1f, e1f, e0f, rel0, mol_index, max0.reshape(128),
        max1.reshape(128), parts0)
    p_flat = jnp.concatenate([p0f, p1f])
    e_flat = jnp.concatenate([e0f, e1f])
    return (contrib.reshape(M, 1),
            ae_f.reshape(N, 1),
            e_flat.reshape(N, 1),
            prob_f.reshape(N, 1),
            p_flat.reshape(N, 1))


# speedup vs baseline: 1.0009x; 1.0009x over previous
import functools

import jax
import jax.numpy as jnp
from jax import lax
from jax.experimental import pallas as pl
from jax.experimental.pallas import tpu as pltpu
from jax.experimental.pallas import tpu_sc as plsc

N = 32768
M = 16
D0 = 128
D1 = 256

BLK = 8192
NB = N // BLK
ROWS = BLK // 128

NSC = 16
H = N // 2
NBH = H // BLK
CHA = H // NSC
CHB = N // NSC
L = 16


def _tc_body(f0, f1, w0, w1, b, p_out, e_out, gmax_out, mscr):
    i = pl.program_id(0)
    dn = (((0,), (1,)), ((), ()))
    acc = lax.dot_general(w0[...], f0[...], dn,
                          preferred_element_type=jnp.float32)
    acc = acc + lax.dot_general(w1[...], f1[...], dn,
                                preferred_element_type=jnp.float32)
    e = acc[0:1, :] + b[0:1, 0:1]
    p = acc[1:2, :]
    p_out[...] = p.reshape(1, 1, BLK)
    e_out[...] = e.reshape(1, 1, BLK)
    bm = jnp.max(p)

    @pl.when(i == 0)
    def _init():
        mscr[...] = jnp.full((1, 128), -jnp.inf, jnp.float32)

    mscr[...] = jnp.maximum(mscr[...], bm)

    @pl.when(i == NBH - 1)
    def _fin():
        gmax_out[...] = mscr[...]


def _tc_pass1(feat0, feat1, w0, w1, bvec, off):
    return pl.pallas_call(
        _tc_body,
        grid=(NBH,),
        in_specs=[
            pl.BlockSpec((BLK, D0), lambda i: (i + off, 0)),
            pl.BlockSpec((BLK, D1), lambda i: (i + off, 0)),
            pl.BlockSpec((D0, 2), lambda i: (0, 0)),
            pl.BlockSpec((D1, 2), lambda i: (0, 0)),
            pl.BlockSpec((8, 128), lambda i: (0, 0)),
        ],
        out_specs=[
            pl.BlockSpec((1, 1, BLK), lambda i: (i, 0, 0)),
            pl.BlockSpec((1, 1, BLK), lambda i: (i, 0, 0)),
            pl.BlockSpec((1, 128), lambda i: (0, 0)),
        ],
        out_shape=[
            jax.ShapeDtypeStruct((NBH, 1, BLK), jnp.float32),
            jax.ShapeDtypeStruct((NBH, 1, BLK), jnp.float32),
            jax.ShapeDtypeStruct((1, 128), jnp.float32),
        ],
        scratch_shapes=[pltpu.VMEM((1, 128), jnp.float32)],
    )(feat0, feat1, w0, w1, bvec)


def _phase_a(p_v, e_v, mol_v, rel_v, maxvec, acc_v, n16):
    z16 = jnp.zeros((L,), jnp.float32)
    for bk in range(4):
        acc_v[pl.ds(bk * 2 * L, L)] = z16
        acc_v[pl.ds(bk * 2 * L + L, L)] = z16

    @plsc.parallel_loop(0, n16, unroll=4)
    def body_a(c):
        sl = pl.ds(c * L, L)
        off = (c % 4) * 2 * M
        mol16 = mol_v[sl] + off
        rel = jnp.exp(p_v[sl] - maxvec)
        rel_v[sl] = rel
        plsc.addupdate_scatter(acc_v, [mol16], rel)
        plsc.addupdate_scatter(acc_v, [mol16 + M], rel * e_v[sl])

    zsum = ((acc_v[pl.ds(0, L)] + acc_v[pl.ds(2 * L, L)])
            + (acc_v[pl.ds(4 * L, L)] + acc_v[pl.ds(6 * L, L)]))
    wsum = ((acc_v[pl.ds(L, L)] + acc_v[pl.ds(3 * L, L)])
            + (acc_v[pl.ds(5 * L, L)] + acc_v[pl.ds(7 * L, L)]))
    acc_v[pl.ds(0, L)] = zsum
    acc_v[pl.ds(L, L)] = wsum


def _sc_a_body(p_hbm, e_hbm, mol_hbm, gmax_hbm,
               rel_hbm, parts_hbm,
               p_v, e_v, mol_v, rel_v, gmax_v, acc_v, sem):
    sid = lax.axis_index("s")
    base = sid * CHA

    c1 = pltpu.async_copy(p_hbm.at[pl.ds(base, CHA)], p_v, sem)
    c2 = pltpu.async_copy(e_hbm.at[pl.ds(base, CHA)], e_v, sem)
    c3 = pltpu.async_copy(mol_hbm.at[pl.ds(base, CHA)], mol_v, sem)
    c4 = pltpu.async_copy(gmax_hbm.at[pl.ds(0, L)], gmax_v, sem)
    c1.wait()
    c2.wait()
    c3.wait()
    c4.wait()

    _phase_a(p_v, e_v, mol_v, rel_v, gmax_v[...], acc_v, CHA // L)

    pltpu.sync_copy(rel_v, rel_hbm.at[pl.ds(base, CHA)])
    pltpu.sync_copy(acc_v.at[pl.ds(0, 2 * M)],
                    parts_hbm.at[pl.ds(sid * 2 * M, 2 * M)])


def _sc_a(p0_flat, e0_flat, mol_index, max0_flat):
    mesh = plsc.VectorSubcoreMesh(core_axis_name="c", subcore_axis_name="s",
                                  num_cores=1)
    fn = pl.kernel(
        _sc_a_body,
        out_type=[
            jax.ShapeDtypeStruct((H,), jnp.float32),
            jax.ShapeDtypeStruct((NSC * 2 * M,), jnp.float32),
        ],
        mesh=mesh,
        compiler_params=pltpu.CompilerParams(needs_layout_passes=False),
        scratch_types=[
            pltpu.VMEM((CHA,), jnp.float32),
            pltpu.VMEM((CHA,), jnp.float32),
            pltpu.VMEM((CHA,), jnp.int32),
            pltpu.VMEM((CHA,), jnp.float32),
            pltpu.VMEM((L,), jnp.float32),
            pltpu.VMEM((8 * L,), jnp.float32),
            pltpu.SemaphoreType.DMA,
        ],
    )
    return fn(p0_flat, e0_flat, mol_index, max0_flat)


def _sc_b_body(p1_hbm, e1_hbm, e0_hbm, rel0_hbm, mol_hbm, max0_hbm, max1_hbm,
               parts0_hbm,
               prob_hbm, ae_hbm, contrib_hbm, parts1_hbm, rel1_hbm,
               pa_v, ea_v, mola_v, rela_v, max0_v, max1_v, acc_v,
               eb_v, molb_v, relb_v, prob_v, ae_v, invz_v, red_v, out16_v,
               sem):
    sid = lax.axis_index("s")
    basea = sid * CHA
    baseb = sid * CHB

    c1 = pltpu.async_copy(p1_hbm.at[pl.ds(basea, CHA)], pa_v, sem)
    c2 = pltpu.async_copy(e1_hbm.at[pl.ds(basea, CHA)], ea_v, sem)
    c3 = pltpu.async_copy(mol_hbm.at[pl.ds(H + basea, CHA)], mola_v, sem)
    c4 = pltpu.async_copy(max0_hbm.at[pl.ds(0, L)], max0_v, sem)
    c5 = pltpu.async_copy(max1_hbm.at[pl.ds(0, L)], max1_v, sem)
    c1.wait()
    c2.wait()
    c3.wait()
    c4.wait()
    c5.wait()

    _phase_a(pa_v, ea_v, mola_v, rela_v, max1_v[...], acc_v, CHA // L)
    pltpu.sync_copy(rela_v, rel1_hbm.at[pl.ds(basea, CHA)])
    pltpu.sync_copy(acc_v.at[pl.ds(0, 2 * M)],
                    parts1_hbm.at[pl.ds(sid * 2 * M, 2 * M)])
    plsc.subcore_barrier()

    c6 = pltpu.async_copy(parts0_hbm, red_v.at[pl.ds(0, NSC * 2 * M)], sem)
    c7 = pltpu.async_copy(parts1_hbm,
                          red_v.at[pl.ds(NSC * 2 * M, NSC * 2 * M)], sem)
    c6.wait()
    c7.wait()

    def red_zw(j, zw):
        z0, w0, z1, w1 = zw
        return (z0 + red_v[pl.ds(j * 2 * M, M)],
                w0 + red_v[pl.ds(j * 2 * M + M, M)],
                z1 + red_v[pl.ds(NSC * 2 * M + j * 2 * M, M)],
                w1 + red_v[pl.ds(NSC * 2 * M + j * 2 * M + M, M)])

    zz = jnp.zeros((L,), jnp.float32)
    z0, w0, z1, w1 = lax.fori_loop(0, NSC, red_zw, (zz, zz, zz, zz))
    g = jnp.maximum(max0_v[...], max1_v[...])
    s0 = jnp.exp(max0_v[...] - g)
    s1 = jnp.exp(max1_v[...] - g)
    zt = s0 * z0 + s1 * z1
    wt = s0 * w0 + s1 * w1
    half0 = sid < (NSC // 2)
    sh = jnp.where(half0, s0, s1)
    invz_v[...] = sh / zt

    @pl.when(sid == 0)
    def _final():
        out16_v[...] = jnp.where(zt > 0.0, wt / zt, 0.0)
        pltpu.sync_copy(out16_v, contrib_hbm)

    relbase = jnp.where(half0, baseb, baseb - H)
    c8 = pltpu.async_copy(mol_hbm.at[pl.ds(baseb, CHB)], molb_v, sem)

    @pl.when(half0)
    def _ld0():
        ca = pltpu.async_copy(rel0_hbm.at[pl.ds(relbase, CHB)], relb_v, sem)
        cb = pltpu.async_copy(e0_hbm.at[pl.ds(relbase, CHB)], eb_v, sem)
        ca.wait()
        cb.wait()

    @pl.when(jnp.logical_not(half0))
    def _ld1():
        ca = pltpu.async_copy(rel1_hbm.at[pl.ds(relbase, CHB)], relb_v, sem)
        cb = pltpu.async_copy(e1_hbm.at[pl.ds(relbase, CHB)], eb_v, sem)
        ca.wait()
        cb.wait()

    c8.wait()

    @plsc.parallel_loop(0, CHB // L, unroll=4)
    def body_b(c):
        sl = pl.ds(c * L, L)
        mol16 = molb_v[sl]
        izg = plsc.load_gather(invz_v, [mol16])
        prob = relb_v[sl] * izg
        prob_v[sl] = prob
        ae_v[sl] = prob * eb_v[sl]

    pltpu.sync_copy(prob_v, prob_hbm.at[pl.ds(baseb, CHB)])
    pltpu.sync_copy(ae_v, ae_hbm.at[pl.ds(baseb, CHB)])


def _sc_b(p1_flat, e1_flat, e0_flat, rel0, mol_index, max0_flat, max1_flat,
          parts0):
    mesh = plsc.VectorSubcoreMesh(core_axis_name="c", subcore_axis_name="s",
                                  num_cores=1)
    fn = pl.kernel(
        _sc_b_body,
        out_type=[
            jax.ShapeDtypeStruct((N,), jnp.float32),
            jax.ShapeDtypeStruct((N,), jnp.float32),
            jax.ShapeDtypeStruct((M,), jnp.float32),
            jax.ShapeDtypeStruct((NSC * 2 * M,), jnp.float32),
            jax.ShapeDtypeStruct((H,), jnp.float32),
        ],
        mesh=mesh,
        compiler_params=pltpu.CompilerParams(needs_layout_passes=False),
        scratch_types=[
            pltpu.VMEM((CHA,), jnp.float32),
            pltpu.VMEM((CHA,), jnp.float32),
            pltpu.VMEM((CHA,), jnp.int32),
            pltpu.VMEM((CHA,), jnp.float32),
            pltpu.VMEM((L,), jnp.float32),
            pltpu.VMEM((L,), jnp.float32),
            pltpu.VMEM((8 * L,), jnp.float32),
            pltpu.VMEM((CHB,), jnp.float32),
            pltpu.VMEM((CHB,), jnp.int32),
            pltpu.VMEM((CHB,), jnp.float32),
            pltpu.VMEM((CHB,), jnp.float32),
            pltpu.VMEM((CHB,), jnp.float32),
            pltpu.VMEM((L,), jnp.float32),
            pltpu.VMEM((2 * NSC * 2 * M,), jnp.float32),
            pltpu.VMEM((L,), jnp.float32),
            pltpu.SemaphoreType.DMA,
        ],
    )
    return fn(p1_flat, e1_flat, e0_flat, rel0, mol_index, max0_flat,
              max1_flat, parts0)


def kernel(feat0, feat1, W_e0, W_e1, b_e1, W_p0, W_p1, mol_index, atom_index,
           n_molecules, n_atoms_max):
    w0 = jnp.concatenate([W_e0, W_p0], axis=1)
    w1 = jnp.concatenate([W_e1, W_p1], axis=1)
    bvec = jnp.broadcast_to(b_e1.reshape(1, 1), (8, 128))
    p0, e0, max0 = _tc_pass1(feat0, feat1, w0, w1, bvec, 0)
    p0f, e0f = p0.reshape(H), e0.reshape(H)
    rel0, parts0 = _sc_a(p0f, e0f, mol_index, max0.reshape(128))
    p1, e1, max1 = _tc_pass1(feat0, feat1, w0, w1, bvec, NBH)
    p1f, e1f = p1.reshape(H), e1.reshape(H)
    prob_f, ae_f, contrib, _pp, _rr = _sc_b(
        p1f, e1f, e0f, rel0, mol_index, max0.reshape(128),
        max1.reshape(128), parts0)
    p_flat = jnp.concatenate([p0f, p1f])
    e_flat = jnp.concatenate([e0f, e1f])
    return (contrib.reshape(M, 1),
            ae_f.reshape(N, 1),
            e_flat.reshape(N, 1),
            prob_f.reshape(N, 1),
            p_flat.reshape(N, 1))


# speedup vs baseline: 1.0508x; 1.0498x over previous
import functools

import jax
import jax.numpy as jnp
from jax import lax
from jax.experimental import pallas as pl
from jax.experimental.pallas import tpu as pltpu
from jax.experimental.pallas import tpu_sc as plsc

N = 32768
M = 16
D0 = 128
D1 = 256

BLK = 8192
NB = N // BLK
ROWS = BLK // 128

NSC = 16
CH = N // NSC
L = 16


def _tc_body(f0, f1, w0, w1, b, p_out, e_out, gmax_out, mscr):
    i = pl.program_id(0)
    dn = (((0,), (1,)), ((), ()))
    acc = lax.dot_general(w0[...], f0[...], dn,
                          preferred_element_type=jnp.float32)
    acc = acc + lax.dot_general(w1[...], f1[...], dn,
                                preferred_element_type=jnp.float32)
    e = acc[0:1, :] + b[0:1, 0:1]
    p = acc[1:2, :]
    p_out[...] = p.reshape(1, 1, BLK)
    e_out[...] = e.reshape(1, 1, BLK)
    bm = jnp.max(p)

    @pl.when(i == 0)
    def _init():
        mscr[...] = jnp.full((1, 128), -jnp.inf, jnp.float32)

    mscr[...] = jnp.maximum(mscr[...], bm)

    @pl.when(i == NB - 1)
    def _fin():
        gmax_out[...] = mscr[...]


def _tc_pass1(feat0, feat1, w0, w1, bvec):
    return pl.pallas_call(
        _tc_body,
        grid=(NB,),
        in_specs=[
            pl.BlockSpec((BLK, D0), lambda i: (i, 0)),
            pl.BlockSpec((BLK, D1), lambda i: (i, 0)),
            pl.BlockSpec((D0, 2), lambda i: (0, 0)),
            pl.BlockSpec((D1, 2), lambda i: (0, 0)),
            pl.BlockSpec((8, 128), lambda i: (0, 0)),
        ],
        out_specs=[
            pl.BlockSpec((1, 1, BLK), lambda i: (i, 0, 0)),
            pl.BlockSpec((1, 1, BLK), lambda i: (i, 0, 0)),
            pl.BlockSpec((1, 128), lambda i: (0, 0)),
        ],
        out_shape=[
            jax.ShapeDtypeStruct((NB, 1, BLK), jnp.float32),
            jax.ShapeDtypeStruct((NB, 1, BLK), jnp.float32),
            jax.ShapeDtypeStruct((1, 128), jnp.float32),
        ],
        scratch_shapes=[pltpu.VMEM((1, 128), jnp.float32)],
    )(feat0, feat1, w0, w1, bvec)


def _sc_body(p_hbm, e_hbm, mol_hbm, gmax_hbm,
             prob_hbm, ae_hbm, contrib_hbm, parts_hbm,
             p_v, e_v, mol_v, rel_v, prob_v, ae_v,
             gmax_v, acc_v, invz_v, red_v, out16_v, sem):
    sid = lax.axis_index("s")
    base = sid * CH

    c1 = pltpu.async_copy(p_hbm.at[pl.ds(base, CH)], p_v, sem)
    c2 = pltpu.async_copy(e_hbm.at[pl.ds(base, CH)], e_v, sem)
    c3 = pltpu.async_copy(mol_hbm.at[pl.ds(base, CH)], mol_v, sem)
    c4 = pltpu.async_copy(gmax_hbm.at[pl.ds(0, L)], gmax_v, sem)
    c1.wait()
    c2.wait()
    c3.wait()
    c4.wait()

    gmax = gmax_v[...]
    z16 = jnp.zeros((L,), jnp.float32)
    for bk in range(4):
        acc_v[pl.ds(bk * 2 * L, L)] = z16
        acc_v[pl.ds(bk * 2 * L + L, L)] = z16

    @plsc.parallel_loop(0, CH // L, unroll=4)
    def body_a(c):
        s = pl.ds(c * L, L)
        off = (c % 4) * 2 * M
        mol16 = mol_v[s] + off
        rel = jnp.exp(p_v[s] - gmax)
        rel_v[s] = rel
        plsc.addupdate_scatter(acc_v, [mol16], rel)
        plsc.addupdate_scatter(acc_v, [mol16 + M], rel * e_v[s])

    zsum = ((acc_v[pl.ds(0, L)] + acc_v[pl.ds(2 * L, L)])
            + (acc_v[pl.ds(4 * L, L)] + acc_v[pl.ds(6 * L, L)]))
    wsum = ((acc_v[pl.ds(L, L)] + acc_v[pl.ds(3 * L, L)])
            + (acc_v[pl.ds(5 * L, L)] + acc_v[pl.ds(7 * L, L)]))
    acc_v[pl.ds(0, L)] = zsum
    acc_v[pl.ds(L, L)] = wsum
    pltpu.sync_copy(acc_v.at[pl.ds(0, 2 * M)],
                    parts_hbm.at[pl.ds(sid * 2 * M, 2 * M)])
    plsc.subcore_barrier()
    pltpu.sync_copy(parts_hbm, red_v)

    def red_zw(j, zw):
        z, w = zw
        return (z + red_v[pl.ds(j * 2 * M, M)],
                w + red_v[pl.ds(j * 2 * M + M, M)])

    ztot, wtot = lax.fori_loop(
        0, NSC, red_zw,
        (jnp.zeros((L,), jnp.float32), jnp.zeros((L,), jnp.float32)))
    invz_v[...] = 1.0 / ztot

    @pl.when(sid == 0)
    def _final():
        out16_v[...] = jnp.where(ztot > 0.0, wtot / ztot, 0.0)
        pltpu.sync_copy(out16_v, contrib_hbm)

    @plsc.parallel_loop(0, CH // L, unroll=4)
    def body_b(c):
        s = pl.ds(c * L, L)
        mol16 = mol_v[s]
        izg = plsc.load_gather(invz_v, [mol16])
        prob = rel_v[s] * izg
        prob_v[s] = prob
        ae_v[s] = prob * e_v[s]

    pltpu.sync_copy(prob_v, prob_hbm.at[pl.ds(base, CH)])
    pltpu.sync_copy(ae_v, ae_hbm.at[pl.ds(base, CH)])


def _sc_pass2(p_flat, e_flat, mol_index, gmax_flat):
    mesh = plsc.VectorSubcoreMesh(core_axis_name="c", subcore_axis_name="s",
                                  num_cores=1)
    fn = pl.kernel(
        _sc_body,
        out_type=[
            jax.ShapeDtypeStruct((N,), jnp.float32),
            jax.ShapeDtypeStruct((N,), jnp.float32),
            jax.ShapeDtypeStruct((M,), jnp.float32),
            jax.ShapeDtypeStruct((NSC * 2 * M,), jnp.float32),
        ],
        mesh=mesh,
        compiler_params=pltpu.CompilerParams(needs_layout_passes=False),
        scratch_types=[
            pltpu.VMEM((CH,), jnp.float32),
            pltpu.VMEM((CH,), jnp.float32),
            pltpu.VMEM((CH,), jnp.int32),
            pltpu.VMEM((CH,), jnp.float32),
            pltpu.VMEM((CH,), jnp.float32),
            pltpu.VMEM((CH,), jnp.float32),
            pltpu.VMEM((L,), jnp.float32),
            pltpu.VMEM((8 * L,), jnp.float32),
            pltpu.VMEM((L,), jnp.float32),
            pltpu.VMEM((NSC * 2 * M,), jnp.float32),
            pltpu.VMEM((L,), jnp.float32),
            pltpu.SemaphoreType.DMA,
        ],
    )
    return fn(p_flat, e_flat, mol_index, gmax_flat)


def kernel(feat0, feat1, W_e0, W_e1, b_e1, W_p0, W_p1, mol_index, atom_index,
           n_molecules, n_atoms_max):
    w0 = jnp.concatenate([W_e0, W_p0], axis=1)
    w1 = jnp.concatenate([W_e1, W_p1], axis=1)
    bvec = jnp.broadcast_to(b_e1.reshape(1, 1), (8, 128))
    p2d, e2d, gmax2d = _tc_pass1(feat0, feat1, w0, w1, bvec)
    p_flat = p2d.reshape(N)
    e_flat = e2d.reshape(N)
    gmax_flat = gmax2d.reshape(128)
    prob_f, ae_f, contrib, _zw = _sc_pass2(p_flat, e_flat, mol_index,
                                           gmax_flat)
    return (contrib.reshape(M, 1),
            ae_f.reshape(N, 1),
            e_flat.reshape(N, 1),
            prob_f.reshape(N, 1),
            p_flat.reshape(N, 1))
